# baseline (speedup 1.0000x reference)
import functools

import jax
import jax.numpy as jnp
from jax import lax
from jax.experimental import pallas as pl
from jax.experimental.pallas import tpu as pltpu
from jax.experimental.pallas import tpu_sc as plsc

VOCAB_ = 100000
HID_ = 128
BATCH_ = 1024
SEQ_ = 50

_TB_ = 8
_SEG_A_ = 16


def _sc_gather(emb, idx_flat):
    info = plsc.get_sparse_core_info()
    nw = info.num_cores * info.num_subcores
    n = idx_flat.shape[0]
    per_w = n // nw
    n_chunks = 2 if per_w <= 800 else 4
    chunk = per_w // n_chunks
    mesh = plsc.VectorSubcoreMesh(core_axis_name="c", subcore_axis_name="s")

    @functools.partial(
        pl.kernel,
        mesh=mesh,
        out_type=jax.ShapeDtypeStruct((n, HID_), jnp.float32),
        scratch_types=[
            pltpu.VMEM((per_w,), jnp.int32),
            pltpu.VMEM((2, chunk, HID_), jnp.float32),
            pltpu.SemaphoreType.DMA,
            pltpu.SemaphoreType.DMA,
            pltpu.SemaphoreType.DMA,
            pltpu.SemaphoreType.DMA,
        ],
    )
    def gather_k(idx_hbm, table_hbm, out_hbm, idx_v, rows_v, sg0, sg1, sw0, sw1):
        wid = lax.axis_index("s") * info.num_cores + lax.axis_index("c")
        base = wid * per_w
        pltpu.sync_copy(idx_hbm.at[pl.ds(base, per_w)], idx_v)
        sg = (sg0, sg1)
        sw = (sw0, sw1)
        pltpu.async_copy(
            table_hbm.at[idx_v.at[pl.ds(0, chunk)]], rows_v.at[0], sg0)
        for c in range(n_chunks):
            b = c % 2
            if c + 1 < n_chunks:
                nb = (c + 1) % 2
                if c + 1 >= 2:
                    pltpu.make_async_copy(
                        rows_v.at[nb],
                        out_hbm.at[pl.ds(base + (c - 1) * chunk, chunk)],
                        sw[nb],
                    ).wait()
                pltpu.async_copy(
                    table_hbm.at[idx_v.at[pl.ds((c + 1) * chunk, chunk)]],
                    rows_v.at[nb], sg[nb])
            pltpu.make_async_copy(
                table_hbm.at[idx_v.at[pl.ds(c * chunk, chunk)]],
                rows_v.at[b], sg[b]).wait()
            pltpu.async_copy(
                rows_v.at[b], out_hbm.at[pl.ds(base + c * chunk, chunk)], sw[b])
        pltpu.make_async_copy(
            rows_v.at[(n_chunks - 2) % 2],
            out_hbm.at[pl.ds(base + (n_chunks - 2) * chunk, chunk)],
            sw[(n_chunks - 2) % 2],
        ).wait()
        pltpu.make_async_copy(
            rows_v.at[(n_chunks - 1) % 2],
            out_hbm.at[pl.ds(base + (n_chunks - 1) * chunk, chunk)],
            sw[(n_chunks - 1) % 2],
        ).wait()

    return gather_k(idx_flat, emb)


def _gru_step(x, h, wih, whh, bi_v, bh_v):
    gi = jnp.dot(x, wih, preferred_element_type=jnp.float32) + bi_v
    gh = jnp.dot(h, whh, preferred_element_type=jnp.float32) + bh_v
    r = jax.nn.sigmoid(gi[:, :HID_] + gh[:, :HID_])
    z = jax.nn.sigmoid(gi[:, HID_:2 * HID_] + gh[:, HID_:2 * HID_])
    nn = jnp.tanh(gi[:, 2 * HID_:] + r * gh[:, 2 * HID_:])
    return (1.0 - z) * nn + z * h


def _gru_seg_a(x3, wih_t, whh_t, bi, bh):
    nblk = _SEG_A_ // _TB_

    def body(x_ref, wih_ref, whh_ref, bi_ref, bh_ref, out_ref, hl_ref, h_ref):
        k = pl.program_id(0)

        @pl.when(k == 0)
        def _init():
            h_ref[...] = jnp.zeros_like(h_ref)

        h = h_ref[...]
        for i in range(_TB_):
            h = _gru_step(x_ref[i], h, wih_ref[...], whh_ref[...],
                          bi_ref[...], bh_ref[...])
            out_ref[:, i, :] = h
        h_ref[...] = h

        @pl.when(k == nblk - 1)
        def _save():
            hl_ref[...] = h

    return pl.pallas_call(
        body,
        grid=(nblk,),
        in_specs=[
            pl.BlockSpec((_TB_, BATCH_, HID_), lambda k: (k, 0, 0)),
            pl.BlockSpec((HID_, 3 * HID_), lambda k: (0, 0)),
            pl.BlockSpec((HID_, 3 * HID_), lambda k: (0, 0)),
            pl.BlockSpec((1, 3 * HID_), lambda k: (0, 0)),
            pl.BlockSpec((1, 3 * HID_), lambda k: (0, 0)),
        ],
        out_specs=[
            pl.BlockSpec((BATCH_, _TB_, HID_), lambda k: (0, k, 0)),
            pl.BlockSpec((BATCH_, HID_), lambda k: (0, 0)),
        ],
        out_shape=[
            jax.ShapeDtypeStruct((BATCH_, _SEG_A_, HID_), jnp.float32),
            jax.ShapeDtypeStruct((BATCH_, HID_), jnp.float32),
        ],
        scratch_shapes=[pltpu.VMEM((BATCH_, HID_), jnp.float32)],
        compiler_params=pltpu.CompilerParams(
            dimension_semantics=("arbitrary",),
        ),
    )(x3, wih_t, whh_t, bi, bh)


def _gru_seg_b(x3, wih_t, whh_t, bi, bh, h0, out_a):
    blk_off = _SEG_A_ // _TB_
    nblk = (SEQ_ + _TB_ - 1) // _TB_
    rem = SEQ_ - (nblk - 1) * _TB_

    def body(x_ref, wih_ref, whh_ref, bi_ref, bh_ref, h0_ref, buf_ref,
             out_ref, h_ref, ob_ref, so_ref, sp):
        k = pl.program_id(0)

        @pl.when(k == 0)
        def _init():
            h_ref[...] = h0_ref[...]
            pltpu.async_copy(
                buf_ref, out_ref.at[:, pl.ds(0, _SEG_A_), :], sp)

        @pl.when(k >= blk_off)
        def _compute():
            b = lax.rem(k, 2)

            @pl.when(k >= blk_off + 2)
            def _drain():
                pltpu.make_async_copy(
                    ob_ref.at[b],
                    out_ref.at[:, pl.ds((k - 2) * _TB_, _TB_), :],
                    so_ref.at[b],
                ).wait()

            h = h_ref[...]
            for i in range(_TB_):
                h = _gru_step(x_ref[i], h, wih_ref[...], whh_ref[...],
                              bi_ref[...], bh_ref[...])
                ob_ref[b, :, i, :] = h
            h_ref[...] = h

            @pl.when(k < nblk - 1)
            def _fire_full():
                pltpu.async_copy(
                    ob_ref.at[b],
                    out_ref.at[:, pl.ds(k * _TB_, _TB_), :],
                    so_ref.at[b])

            @pl.when(k == nblk - 1)
            def _tail():
                pltpu.async_copy(
                    ob_ref.at[b, :, pl.ds(0, rem), :],
                    out_ref.at[:, pl.ds((nblk - 1) * _TB_, rem), :],
                    so_ref.at[b])
                pltpu.make_async_copy(
                    ob_ref.at[1 - b],
                    out_ref.at[:, pl.ds((nblk - 2) * _TB_, _TB_), :],
                    so_ref.at[1 - b],
                ).wait()
                pltpu.make_async_copy(
                    ob_ref.at[b, :, pl.ds(0, rem), :],
                    out_ref.at[:, pl.ds((nblk - 1) * _TB_, rem), :],
                    so_ref.at[b],
                ).wait()
                pltpu.make_async_copy(
                    buf_ref, out_ref.at[:, pl.ds(0, _SEG_A_), :], sp,
                ).wait()

    return pl.pallas_call(
        body,
        grid=(nblk,),
        in_specs=[
            pl.BlockSpec((_TB_, BATCH_, HID_),
                         lambda k: (jnp.maximum(k - blk_off, 0), 0, 0)),
            pl.BlockSpec((HID_, 3 * HID_), lambda k: (0, 0)),
            pl.BlockSpec((HID_, 3 * HID_), lambda k: (0, 0)),
            pl.BlockSpec((1, 3 * HID_), lambda k: (0, 0)),
            pl.BlockSpec((1, 3 * HID_), lambda k: (0, 0)),
            pl.BlockSpec((BATCH_, HID_), lambda k: (0, 0)),
            pl.BlockSpec(memory_space=pltpu.MemorySpace.HBM),
        ],
        out_specs=pl.BlockSpec(memory_space=pltpu.MemorySpace.HBM),
        out_shape=jax.ShapeDtypeStruct((BATCH_, SEQ_, HID_), jnp.float32),
        scratch_shapes=[
            pltpu.VMEM((BATCH_, HID_), jnp.float32),
            pltpu.VMEM((2, BATCH_, _TB_, HID_), jnp.float32),
            pltpu.SemaphoreType.DMA((2,)),
            pltpu.SemaphoreType.DMA,
        ],
        compiler_params=pltpu.CompilerParams(
            dimension_semantics=("arbitrary",),
        ),
    )(x3, wih_t, whh_t, bi, bh, h0, out_a)


def kernel(T, emb, W_ih, W_hh, b_ih, b_hh):
    idx = T.T.reshape(-1).astype(jnp.int32)
    na = _SEG_A_ * BATCH_
    x_a = _sc_gather(emb, idx[:na]).reshape(_SEG_A_, BATCH_, HID_)
    x_b = _sc_gather(emb, idx[na:]).reshape(SEQ_ - _SEG_A_, BATCH_, HID_)
    wih_t = W_ih.T
    whh_t = W_hh.T
    bi = b_ih.reshape(1, -1)
    bh = b_hh.reshape(1, -1)
    out_a, h_mid = _gru_seg_a(x_a, wih_t, whh_t, bi, bh)
    outputs = _gru_seg_b(x_b, wih_t, whh_t, bi, bh, h_mid, out_a)
    hidden = outputs[:, SEQ_ - 1][None]
    return (outputs, hidden)


# speedup vs baseline: 3.0227x; 3.0227x over previous
import functools

import jax
import jax.numpy as jnp
from jax import lax
from jax.experimental import pallas as pl
from jax.experimental.pallas import tpu as pltpu
from jax.experimental.pallas import tpu_sc as plsc

VOCAB_ = 100000
HID_ = 128
BATCH_ = 1024
SEQ_ = 50

_TB_ = 8
_SEG_A_ = 16


def _sc_gather(emb, idx_flat):
    info = plsc.get_sparse_core_info()
    nw = info.num_cores * info.num_subcores
    n = idx_flat.shape[0]
    per_w = n // nw
    n_chunks = 2 if per_w <= 800 else 4
    chunk = per_w // n_chunks
    mesh = plsc.VectorSubcoreMesh(core_axis_name="c", subcore_axis_name="s")

    @functools.partial(
        pl.kernel,
        mesh=mesh,
        out_type=jax.ShapeDtypeStruct((n, HID_), jnp.float32),
        scratch_types=[
            pltpu.VMEM((per_w,), jnp.int32),
            pltpu.VMEM((2, chunk, HID_), jnp.float32),
            pltpu.SemaphoreType.DMA,
            pltpu.SemaphoreType.DMA,
            pltpu.SemaphoreType.DMA,
            pltpu.SemaphoreType.DMA,
        ],
    )
    def gather_k(idx_hbm, table_hbm, out_hbm, idx_v, rows_v, sg0, sg1, sw0, sw1):
        wid = lax.axis_index("s") * info.num_cores + lax.axis_index("c")
        base = wid * per_w
        pltpu.sync_copy(idx_hbm.at[pl.ds(base, per_w)], idx_v)
        sg = (sg0, sg1)
        sw = (sw0, sw1)
        pltpu.async_copy(
            table_hbm.at[idx_v.at[pl.ds(0, chunk)]], rows_v.at[0], sg0)
        for c in range(n_chunks):
            b = c % 2
            if c + 1 < n_chunks:
                nb = (c + 1) % 2
                if c + 1 >= 2:
                    pltpu.make_async_copy(
                        rows_v.at[nb],
                        out_hbm.at[pl.ds(base + (c - 1) * chunk, chunk)],
                        sw[nb],
                    ).wait()
                pltpu.async_copy(
                    table_hbm.at[idx_v.at[pl.ds((c + 1) * chunk, chunk)]],
                    rows_v.at[nb], sg[nb])
            pltpu.make_async_copy(
                table_hbm.at[idx_v.at[pl.ds(c * chunk, chunk)]],
                rows_v.at[b], sg[b]).wait()
            pltpu.async_copy(
                rows_v.at[b], out_hbm.at[pl.ds(base + c * chunk, chunk)], sw[b])
        pltpu.make_async_copy(
            rows_v.at[(n_chunks - 2) % 2],
            out_hbm.at[pl.ds(base + (n_chunks - 2) * chunk, chunk)],
            sw[(n_chunks - 2) % 2],
        ).wait()
        pltpu.make_async_copy(
            rows_v.at[(n_chunks - 1) % 2],
            out_hbm.at[pl.ds(base + (n_chunks - 1) * chunk, chunk)],
            sw[(n_chunks - 1) % 2],
        ).wait()

    return gather_k(idx_flat, emb)


def _gru_step(x, h, wih, whh, bi_v, bh_v):
    gi = jnp.dot(x, wih, preferred_element_type=jnp.float32) + bi_v
    gh = jnp.dot(h, whh, preferred_element_type=jnp.float32) + bh_v
    r = jax.nn.sigmoid(gi[:, :HID_] + gh[:, :HID_])
    z = jax.nn.sigmoid(gi[:, HID_:2 * HID_] + gh[:, HID_:2 * HID_])
    nn = jnp.tanh(gi[:, 2 * HID_:] + r * gh[:, 2 * HID_:])
    return (1.0 - z) * nn + z * h


def _gru_seg_a(x3, wih_t, whh_t, bi, bh):
    nblk = _SEG_A_ // _TB_

    def body(x_ref, wih_ref, whh_ref, bi_ref, bh_ref, out_ref, hl_ref, h_ref):
        k = pl.program_id(0)

        @pl.when(k == 0)
        def _init():
            h_ref[...] = jnp.zeros_like(h_ref)

        h = h_ref[...]
        for i in range(_TB_):
            h = _gru_step(x_ref[i], h, wih_ref[...], whh_ref[...],
                          bi_ref[...], bh_ref[...])
            out_ref[:, i, :] = h
        h_ref[...] = h

        @pl.when(k == nblk - 1)
        def _save():
            hl_ref[...] = h

    return pl.pallas_call(
        body,
        grid=(nblk,),
        in_specs=[
            pl.BlockSpec((_TB_, BATCH_, HID_), lambda k: (k, 0, 0)),
            pl.BlockSpec((HID_, 3 * HID_), lambda k: (0, 0)),
            pl.BlockSpec((HID_, 3 * HID_), lambda k: (0, 0)),
            pl.BlockSpec((1, 3 * HID_), lambda k: (0, 0)),
            pl.BlockSpec((1, 3 * HID_), lambda k: (0, 0)),
        ],
        out_specs=[
            pl.BlockSpec((BATCH_, _TB_, HID_), lambda k: (0, k, 0)),
            pl.BlockSpec((BATCH_, HID_), lambda k: (0, 0)),
        ],
        out_shape=[
            jax.ShapeDtypeStruct((BATCH_, SEQ_, HID_), jnp.float32),
            jax.ShapeDtypeStruct((BATCH_, HID_), jnp.float32),
        ],
        scratch_shapes=[pltpu.VMEM((BATCH_, HID_), jnp.float32)],
        compiler_params=pltpu.CompilerParams(
            dimension_semantics=("arbitrary",),
        ),
    )(x3, wih_t, whh_t, bi, bh)


def _gru_seg_b(x3, wih_t, whh_t, bi, bh, h0, out_a):
    seg_len = SEQ_ - _SEG_A_
    blk_off = _SEG_A_ // _TB_
    nblk = (seg_len + _TB_ - 1) // _TB_

    def body(x_ref, wih_ref, whh_ref, bi_ref, bh_ref, h0_ref, buf_ref,
             out_ref, h_ref):
        k = pl.program_id(0)

        @pl.when(k == 0)
        def _init():
            h_ref[...] = h0_ref[...]

        h = h_ref[...]
        for i in range(_TB_):
            h = _gru_step(x_ref[i], h, wih_ref[...], whh_ref[...],
                          bi_ref[...], bh_ref[...])
            out_ref[:, i, :] = h
        h_ref[...] = h

    return pl.pallas_call(
        body,
        grid=(nblk,),
        in_specs=[
            pl.BlockSpec((_TB_, BATCH_, HID_), lambda k: (k, 0, 0)),
            pl.BlockSpec((HID_, 3 * HID_), lambda k: (0, 0)),
            pl.BlockSpec((HID_, 3 * HID_), lambda k: (0, 0)),
            pl.BlockSpec((1, 3 * HID_), lambda k: (0, 0)),
            pl.BlockSpec((1, 3 * HID_), lambda k: (0, 0)),
            pl.BlockSpec((BATCH_, HID_), lambda k: (0, 0)),
            pl.BlockSpec(memory_space=pltpu.MemorySpace.HBM),
        ],
        out_specs=pl.BlockSpec(
            (BATCH_, _TB_, HID_), lambda k: (0, k + blk_off, 0)),
        out_shape=jax.ShapeDtypeStruct((BATCH_, SEQ_, HID_), jnp.float32),
        scratch_shapes=[pltpu.VMEM((BATCH_, HID_), jnp.float32)],
        input_output_aliases={6: 0},
        compiler_params=pltpu.CompilerParams(
            dimension_semantics=("arbitrary",),
        ),
    )(x3, wih_t, whh_t, bi, bh, h0, out_a)


def kernel(T, emb, W_ih, W_hh, b_ih, b_hh):
    idx = T.T.reshape(-1).astype(jnp.int32)
    na = _SEG_A_ * BATCH_
    x_a = _sc_gather(emb, idx[:na]).reshape(_SEG_A_, BATCH_, HID_)
    x_b = _sc_gather(emb, idx[na:]).reshape(SEQ_ - _SEG_A_, BATCH_, HID_)
    wih_t = W_ih.T
    whh_t = W_hh.T
    bi = b_ih.reshape(1, -1)
    bh = b_hh.reshape(1, -1)
    out_a, h_mid = _gru_seg_a(x_a, wih_t, whh_t, bi, bh)
    outputs = _gru_seg_b(x_b, wih_t, whh_t, bi, bh, h_mid, out_a)
    hidden = outputs[:, SEQ_ - 1][None]
    return (outputs, hidden)
